# baseline (speedup 1.0000x reference)
import functools

import jax
import jax.numpy as jnp
from jax import lax
from jax.experimental import pallas as pl
from jax.experimental.pallas import tpu as pltpu
from jax.experimental.pallas import tpu_sc as plsc

L = 16


def _pack_body(v3_ref, out_ref):
    v = v3_ref[:]
    v0 = v[:, :-1, :]
    dv = v[:, 1:, :] - v0
    h0 = lax.bitcast_convert_type(v0.astype(jnp.bfloat16), jnp.uint16)
    h1 = lax.bitcast_convert_type(dv.astype(jnp.bfloat16), jnp.uint16)
    packed = (h1.astype(jnp.uint32) << 16) | h0.astype(jnp.uint32)
    out_ref[:] = lax.bitcast_convert_type(packed, jnp.int32)


def _pack_table(v3):
    D, K, O = v3.shape
    pt3 = pl.pallas_call(
        _pack_body,
        out_shape=jax.ShapeDtypeStruct((D, K - 1, O), jnp.int32),
    )(v3)
    return pt3.reshape(D * (K - 1), O)


def _sc_spline_fn(B, D, O, K, NB, GCH, NBG):
    NG = NB // NBG
    NCHG = NBG * (D // GCH)
    NOC = O // L
    scale = (K - 1) * 0.5
    mesh = plsc.VectorSubcoreMesh(core_axis_name="c", subcore_axis_name="s")

    def body(x_hbm, pt_hbm, out_hbm, xw_v, idx_v, g0, g1, stage_v,
             sem0, sem1):
        col = [lax.iota(jnp.int32, L) + oc * L for oc in range(NOC)]
        wid = lax.axis_index("s") * 2 + lax.axis_index("c")
        b0 = wid * NB

        def splat(v):
            return jnp.full((L,), v, jnp.int32)

        def gather_dma(ci, g, sem):
            cis = jnp.minimum(ci, NCHG - 1)
            return pltpu.make_async_copy(
                pt_hbm.at[idx_v.at[pl.ds(cis * GCH, GCH)]], g, sem)

        def chunk_compute(bl, q, g, acc):
            def d_body(dl, acc):
                wsp = plsc.load_gather(xw_v, [splat(bl * D + q * GCH + dl)])
                new = []
                for oc in range(NOC):
                    p = g[dl, pl.ds(oc * L, L)]
                    v0 = plsc.bitcast(p << 16, jnp.float32)
                    dv = plsc.bitcast(p & jnp.int32(-65536), jnp.float32)
                    new.append(acc[oc] + v0 + wsp * dv)
                return tuple(new)
            return lax.fori_loop(0, GCH, d_body, acc)

        nq = D // GCH

        def group_body(gi, _):
            gb0 = b0 + gi * NBG
            pltpu.sync_copy(x_hbm.at[pl.ds(gb0 * D, NBG * D)], xw_v)

            def prep_b(bl, _):
                for c in range(D // L):
                    fb = splat(bl * D + c * L) + col[0]
                    xx = plsc.load_gather(xw_v, [fb])
                    t = jnp.clip(xx, -1.0, 1.0) * scale + scale
                    li = jnp.clip(t.astype(jnp.int32), 0, K - 2)
                    w = t - li.astype(jnp.float32)
                    rows = (lax.iota(jnp.int32, L) + c * L) * (K - 1) + li
                    plsc.store_scatter(idx_v, [fb], rows)
                    plsc.store_scatter(xw_v, [fb], w)
                return 0
            lax.fori_loop(0, NBG, prep_b, 0)

            gather_dma(jnp.int32(0), g0, sem0).start()

            def b_body(bl, _):
                ci0 = bl * nq
                acc = tuple(jnp.zeros((L,), jnp.float32) for _ in range(NOC))
                for q in range(nq):
                    g, sem = (g0, sem0) if q % 2 == 0 else (g1, sem1)
                    gn, semn = (g1, sem1) if q % 2 == 0 else (g0, sem0)
                    gather_dma(ci0 + q + 1, gn, semn).start()
                    gather_dma(ci0 + q, g, sem).wait()
                    acc = chunk_compute(bl, q, g, acc)
                for oc in range(NOC):
                    stage_v[pl.ds(oc * L, L)] = acc[oc]
                pltpu.sync_copy(stage_v, out_hbm.at[gb0 + bl])
                return 0
            lax.fori_loop(0, NBG, b_body, 0)

            gather_dma(jnp.int32(NCHG - 1), g0 if nq % 2 == 0 else g1,
                       sem0 if nq % 2 == 0 else sem1).wait()
            return 0
        lax.fori_loop(0, NG, group_body, 0)

    return functools.partial(
        pl.kernel,
        mesh=mesh,
        compiler_params=pltpu.CompilerParams(needs_layout_passes=False),
        out_type=jax.ShapeDtypeStruct((B, O), jnp.float32),
        scratch_types=[
            pltpu.VMEM((NBG * D,), jnp.float32),
            pltpu.VMEM((NBG * D,), jnp.int32),
            pltpu.VMEM((GCH, O), jnp.int32),
            pltpu.VMEM((GCH, O), jnp.int32),
            pltpu.VMEM((O,), jnp.float32),
            pltpu.SemaphoreType.DMA,
            pltpu.SemaphoreType.DMA,
        ],
    )(body)


def _tc_spline_body(x_ref, vt_ref, out_ref, *, K):
    bm, D = x_ref.shape
    xc = jnp.clip(x_ref[:], -1.0, 1.0)
    t = (xc + 1.0) * ((K - 1) * 0.5)
    lf = jnp.clip(jnp.floor(t), 0.0, K - 2.0)
    w = t - lf
    li = lf.astype(jnp.int32)
    kk = lax.broadcasted_iota(jnp.int32, (bm, D, K), 2)
    l3 = li[:, :, None]
    w3 = w[:, :, None]
    W = jnp.where(kk == l3, 1.0 - w3, jnp.where(kk == l3 + 1, w3, 0.0))
    Wb = W.reshape(bm, D * K).astype(jnp.bfloat16)
    out_ref[:] = jnp.dot(Wb, vt_ref[:], preferred_element_type=jnp.float32)


def _tc_spline(x_part, vt, K, bm=256):
    BT, D = x_part.shape
    O = vt.shape[1]
    return pl.pallas_call(
        functools.partial(_tc_spline_body, K=K),
        grid=(BT // bm,),
        in_specs=[
            pl.BlockSpec((bm, D), lambda i: (i, 0)),
            pl.BlockSpec((D * K, O), lambda i: (0, 0)),
        ],
        out_specs=pl.BlockSpec((bm, O), lambda i: (i, 0)),
        out_shape=jax.ShapeDtypeStruct((BT, O), jnp.float32),
    )(x_part, vt.astype(jnp.bfloat16))


def _finish_body(sp_ref, x_ref, swt_ref, sb_ref, out_ref):
    xc = jnp.clip(x_ref[:], -1.0, 1.0)
    out_ref[:] = (sp_ref[:]
                  + jnp.dot(xc, swt_ref[:], preferred_element_type=jnp.float32)
                  + sb_ref[:])


def kernel(x, values, skip_w, skip_b):
    B, D = x.shape
    O, _, K = values.shape
    NW = 32
    GCH = 64
    BT = 2560
    BS = B - BT
    NB = BS // NW

    v3 = values.transpose(1, 2, 0)
    pt = _pack_table(v3)
    spline_sc = _sc_spline_fn(BS, D, O, K, NB, GCH, NB)(
        x[BT:].reshape(-1), pt)
    vt = values.reshape(O, D * K).T
    spline_tc = _tc_spline(x[:BT], vt, K)
    spline = jnp.concatenate([spline_tc, spline_sc], axis=0)

    swt = skip_w.T
    sb2 = skip_b[None, :]
    bm = min(B, 512)
    return pl.pallas_call(
        _finish_body,
        grid=(B // bm,),
        in_specs=[
            pl.BlockSpec((bm, O), lambda i: (i, 0)),
            pl.BlockSpec((bm, D), lambda i: (i, 0)),
            pl.BlockSpec((D, O), lambda i: (0, 0)),
            pl.BlockSpec((1, O), lambda i: (0, 0)),
        ],
        out_specs=pl.BlockSpec((bm, O), lambda i: (i, 0)),
        out_shape=jax.ShapeDtypeStruct((B, O), jnp.float32),
    )(spline, x, swt, sb2)


# speedup vs baseline: 1.0727x; 1.0727x over previous
import functools

import jax
import jax.numpy as jnp
from jax import lax
from jax.experimental import pallas as pl
from jax.experimental.pallas import tpu as pltpu
from jax.experimental.pallas import tpu_sc as plsc

L = 16


def _pack_body(v3_ref, out_ref):
    v = v3_ref[:]
    v0 = v[:, :-1, :]
    dv = v[:, 1:, :] - v0
    h0 = lax.bitcast_convert_type(v0.astype(jnp.bfloat16), jnp.uint16)
    h1 = lax.bitcast_convert_type(dv.astype(jnp.bfloat16), jnp.uint16)
    packed = (h1.astype(jnp.uint32) << 16) | h0.astype(jnp.uint32)
    out_ref[:] = lax.bitcast_convert_type(packed, jnp.int32)


def _pack_table(v3):
    D, K, O = v3.shape
    pt3 = pl.pallas_call(
        _pack_body,
        out_shape=jax.ShapeDtypeStruct((D, K - 1, O), jnp.int32),
    )(v3)
    return pt3.reshape(D * (K - 1), O)


def _sc_spline_fn(B, D, O, K, NB, GCH, NBG):
    NG = NB // NBG
    NCHG = NBG * (D // GCH)
    NOC = O // L
    scale = (K - 1) * 0.5
    mesh = plsc.VectorSubcoreMesh(core_axis_name="c", subcore_axis_name="s")

    def body(x_hbm, pt_hbm, out_hbm, xw_v, idx_v, g0, g1, stage_v,
             sem0, sem1):
        col = [lax.iota(jnp.int32, L) + oc * L for oc in range(NOC)]
        wid = lax.axis_index("s") * 2 + lax.axis_index("c")
        b0 = wid * NB

        def splat(v):
            return jnp.full((L,), v, jnp.int32)

        def gather_dma(ci, g, sem):
            cis = jnp.minimum(ci, NCHG - 1)
            return pltpu.make_async_copy(
                pt_hbm.at[idx_v.at[pl.ds(cis * GCH, GCH)]], g, sem)

        def chunk_compute(bl, q, g, acc):
            def d_body(dl, acc):
                wsp = plsc.load_gather(xw_v, [splat(bl * D + q * GCH + dl)])
                new = []
                for oc in range(NOC):
                    p = g[dl, pl.ds(oc * L, L)]
                    v0 = plsc.bitcast(p << 16, jnp.float32)
                    dv = plsc.bitcast(p & jnp.int32(-65536), jnp.float32)
                    new.append(acc[oc] + v0 + wsp * dv)
                return tuple(new)
            return lax.fori_loop(0, GCH, d_body, acc)

        nq = D // GCH

        def group_body(gi, _):
            gb0 = b0 + gi * NBG
            pltpu.sync_copy(x_hbm.at[pl.ds(gb0 * D, NBG * D)], xw_v)

            def prep_b(bl, _):
                for c in range(D // L):
                    fb = splat(bl * D + c * L) + col[0]
                    xx = plsc.load_gather(xw_v, [fb])
                    t = jnp.clip(xx, -1.0, 1.0) * scale + scale
                    li = jnp.clip(t.astype(jnp.int32), 0, K - 2)
                    w = t - li.astype(jnp.float32)
                    rows = (lax.iota(jnp.int32, L) + c * L) * (K - 1) + li
                    plsc.store_scatter(idx_v, [fb], rows)
                    plsc.store_scatter(xw_v, [fb], w)
                return 0
            lax.fori_loop(0, NBG, prep_b, 0)

            gather_dma(jnp.int32(0), g0, sem0).start()

            def b_body(bl, _):
                ci0 = bl * nq
                acc = tuple(jnp.zeros((L,), jnp.float32) for _ in range(NOC))
                for q in range(nq):
                    g, sem = (g0, sem0) if q % 2 == 0 else (g1, sem1)
                    gn, semn = (g1, sem1) if q % 2 == 0 else (g0, sem0)
                    gather_dma(ci0 + q + 1, gn, semn).start()
                    gather_dma(ci0 + q, g, sem).wait()
                    acc = chunk_compute(bl, q, g, acc)
                for oc in range(NOC):
                    stage_v[pl.ds(oc * L, L)] = acc[oc]
                pltpu.sync_copy(stage_v, out_hbm.at[gb0 + bl])
                return 0
            lax.fori_loop(0, NBG, b_body, 0)

            gather_dma(jnp.int32(NCHG - 1), g0 if nq % 2 == 0 else g1,
                       sem0 if nq % 2 == 0 else sem1).wait()
            return 0
        lax.fori_loop(0, NG, group_body, 0)

    return functools.partial(
        pl.kernel,
        mesh=mesh,
        compiler_params=pltpu.CompilerParams(needs_layout_passes=False),
        out_type=jax.ShapeDtypeStruct((B, O), jnp.float32),
        scratch_types=[
            pltpu.VMEM((NBG * D,), jnp.float32),
            pltpu.VMEM((NBG * D,), jnp.int32),
            pltpu.VMEM((GCH, O), jnp.int32),
            pltpu.VMEM((GCH, O), jnp.int32),
            pltpu.VMEM((O,), jnp.float32),
            pltpu.SemaphoreType.DMA,
            pltpu.SemaphoreType.DMA,
        ],
    )(body)


def _tc_spline_body(x_ref, vt_ref, out_ref, *, K):
    bm, D = x_ref.shape
    xc = jnp.clip(x_ref[:], -1.0, 1.0)
    t = (xc + 1.0) * ((K - 1) * 0.5)
    kk = lax.broadcasted_iota(jnp.int32, (bm, D, K), 2).astype(jnp.float32)
    W = jnp.maximum(1.0 - jnp.abs(t[:, :, None] - kk), 0.0)
    out_ref[:] = jnp.dot(W.reshape(bm, D * K), vt_ref[:],
                         preferred_element_type=jnp.float32)


def _tc_spline(x_part, vt, K, bm=256):
    BT, D = x_part.shape
    O = vt.shape[1]
    return pl.pallas_call(
        functools.partial(_tc_spline_body, K=K),
        grid=(BT // bm,),
        in_specs=[
            pl.BlockSpec((bm, D), lambda i: (i, 0)),
            pl.BlockSpec((D * K, O), lambda i: (0, 0)),
        ],
        out_specs=pl.BlockSpec((bm, O), lambda i: (i, 0)),
        out_shape=jax.ShapeDtypeStruct((BT, O), jnp.float32),
    )(x_part, vt)


def _finish_body(sp_ref, x_ref, swt_ref, sb_ref, out_ref):
    xc = jnp.clip(x_ref[:], -1.0, 1.0)
    out_ref[:] = (sp_ref[:]
                  + jnp.dot(xc, swt_ref[:], preferred_element_type=jnp.float32)
                  + sb_ref[:])


def kernel(x, values, skip_w, skip_b):
    B, D = x.shape
    O, _, K = values.shape
    NW = 32
    GCH = 64
    BT = 2560
    BS = B - BT
    NB = BS // NW

    v3 = values.transpose(1, 2, 0)
    pt = _pack_table(v3)
    spline_sc = _sc_spline_fn(BS, D, O, K, NB, GCH, NB)(
        x[BT:].reshape(-1), pt)
    vt = values.reshape(O, D * K).T
    spline_tc = _tc_spline(x[:BT], vt, K)
    spline = jnp.concatenate([spline_tc, spline_sc], axis=0)

    swt = skip_w.T
    sb2 = skip_b[None, :]
    bm = min(B, 512)
    return pl.pallas_call(
        _finish_body,
        grid=(B // bm,),
        in_specs=[
            pl.BlockSpec((bm, O), lambda i: (i, 0)),
            pl.BlockSpec((bm, D), lambda i: (i, 0)),
            pl.BlockSpec((D, O), lambda i: (0, 0)),
            pl.BlockSpec((1, O), lambda i: (0, 0)),
        ],
        out_specs=pl.BlockSpec((bm, O), lambda i: (i, 0)),
        out_shape=jax.ShapeDtypeStruct((B, O), jnp.float32),
    )(spline, x, swt, sb2)


# speedup vs baseline: 1.0904x; 1.0165x over previous
import functools

import jax
import jax.numpy as jnp
from jax import lax
from jax.experimental import pallas as pl
from jax.experimental.pallas import tpu as pltpu
from jax.experimental.pallas import tpu_sc as plsc

L = 16


def _pack_body(v3_ref, out_ref):
    v = v3_ref[:]
    v0 = v[:, :-1, :]
    dv = v[:, 1:, :] - v0
    h0 = lax.bitcast_convert_type(v0.astype(jnp.bfloat16), jnp.uint16)
    h1 = lax.bitcast_convert_type(dv.astype(jnp.bfloat16), jnp.uint16)
    packed = (h1.astype(jnp.uint32) << 16) | h0.astype(jnp.uint32)
    out_ref[:] = lax.bitcast_convert_type(packed, jnp.int32)


def _pack_table(v3):
    D, K, O = v3.shape
    pt3 = pl.pallas_call(
        _pack_body,
        out_shape=jax.ShapeDtypeStruct((D, K - 1, O), jnp.int32),
    )(v3)
    return pt3.reshape(D * (K - 1), O)


def _sc_spline_fn(B, D, O, K, NB, GCH, NBG):
    NG = NB // NBG
    NCHG = NBG * (D // GCH)
    NOC = O // L
    scale = (K - 1) * 0.5
    mesh = plsc.VectorSubcoreMesh(core_axis_name="c", subcore_axis_name="s")

    def body(x_hbm, pt_hbm, out_hbm, xw_v, idx_v, g0, g1, stage_v,
             sem0, sem1):
        col = [lax.iota(jnp.int32, L) + oc * L for oc in range(NOC)]
        wid = lax.axis_index("s") * 2 + lax.axis_index("c")
        b0 = wid * NB

        def splat(v):
            return jnp.full((L,), v, jnp.int32)

        def gather_dma(ci, g, sem):
            cis = jnp.minimum(ci, NCHG - 1)
            return pltpu.make_async_copy(
                pt_hbm.at[idx_v.at[pl.ds(cis * GCH, GCH)]], g, sem)

        def chunk_compute(bl, q, g, acc):
            def d_body(dl, acc):
                wsp = plsc.load_gather(xw_v, [splat(bl * D + q * GCH + dl)])
                new = []
                for oc in range(NOC):
                    p = g[dl, pl.ds(oc * L, L)]
                    v0 = plsc.bitcast(p << 16, jnp.float32)
                    dv = plsc.bitcast(p & jnp.int32(-65536), jnp.float32)
                    new.append(acc[oc] + v0 + wsp * dv)
                return tuple(new)
            return lax.fori_loop(0, GCH, d_body, acc)

        nq = D // GCH

        def group_body(gi, _):
            gb0 = b0 + gi * NBG
            pltpu.sync_copy(x_hbm.at[pl.ds(gb0 * D, NBG * D)], xw_v)

            def prep_b(bl, _):
                for c in range(D // L):
                    fb = splat(bl * D + c * L) + col[0]
                    xx = plsc.load_gather(xw_v, [fb])
                    t = jnp.clip(xx, -1.0, 1.0) * scale + scale
                    li = jnp.clip(t.astype(jnp.int32), 0, K - 2)
                    w = t - li.astype(jnp.float32)
                    rows = (lax.iota(jnp.int32, L) + c * L) * (K - 1) + li
                    plsc.store_scatter(idx_v, [fb], rows)
                    plsc.store_scatter(xw_v, [fb], w)
                return 0
            lax.fori_loop(0, NBG, prep_b, 0)

            gather_dma(jnp.int32(0), g0, sem0).start()

            def b_body(bl, _):
                ci0 = bl * nq
                acc = tuple(jnp.zeros((L,), jnp.float32) for _ in range(NOC))
                for q in range(nq):
                    g, sem = (g0, sem0) if q % 2 == 0 else (g1, sem1)
                    gn, semn = (g1, sem1) if q % 2 == 0 else (g0, sem0)
                    gather_dma(ci0 + q + 1, gn, semn).start()
                    gather_dma(ci0 + q, g, sem).wait()
                    acc = chunk_compute(bl, q, g, acc)
                for oc in range(NOC):
                    stage_v[pl.ds(oc * L, L)] = acc[oc]
                pltpu.sync_copy(stage_v, out_hbm.at[gb0 + bl])
                return 0
            lax.fori_loop(0, NBG, b_body, 0)

            gather_dma(jnp.int32(NCHG - 1), g0 if nq % 2 == 0 else g1,
                       sem0 if nq % 2 == 0 else sem1).wait()
            return 0
        lax.fori_loop(0, NG, group_body, 0)

    return functools.partial(
        pl.kernel,
        mesh=mesh,
        compiler_params=pltpu.CompilerParams(needs_layout_passes=False),
        out_type=jax.ShapeDtypeStruct((B, O), jnp.float32),
        scratch_types=[
            pltpu.VMEM((NBG * D,), jnp.float32),
            pltpu.VMEM((NBG * D,), jnp.int32),
            pltpu.VMEM((GCH, O), jnp.int32),
            pltpu.VMEM((GCH, O), jnp.int32),
            pltpu.VMEM((O,), jnp.float32),
            pltpu.SemaphoreType.DMA,
            pltpu.SemaphoreType.DMA,
        ],
    )(body)


def _tc_spline_body(x_ref, vt_ref, swt_ref, sb_ref, out_ref, *, K):
    bm, D = x_ref.shape
    xc = jnp.clip(x_ref[:], -1.0, 1.0)
    t = (xc + 1.0) * ((K - 1) * 0.5)
    kk = lax.broadcasted_iota(jnp.int32, (bm, D, K), 2).astype(jnp.float32)
    W = jnp.maximum(1.0 - jnp.abs(t[:, :, None] - kk), 0.0)
    y = jnp.dot(W.reshape(bm, D * K), vt_ref[:],
                preferred_element_type=jnp.float32)
    y = y + jnp.dot(xc, swt_ref[:], preferred_element_type=jnp.float32)
    out_ref[:] = y + sb_ref[:]


def _tc_spline(x_part, vt, swt, sb2, K, bm=256):
    BT, D = x_part.shape
    O = vt.shape[1]
    return pl.pallas_call(
        functools.partial(_tc_spline_body, K=K),
        grid=(BT // bm,),
        in_specs=[
            pl.BlockSpec((bm, D), lambda i: (i, 0)),
            pl.BlockSpec((D * K, O), lambda i: (0, 0)),
            pl.BlockSpec((D, O), lambda i: (0, 0)),
            pl.BlockSpec((1, O), lambda i: (0, 0)),
        ],
        out_specs=pl.BlockSpec((bm, O), lambda i: (i, 0)),
        out_shape=jax.ShapeDtypeStruct((BT, O), jnp.float32),
    )(x_part, vt, swt, sb2)


def _finish_body(sp_ref, x_ref, swt_ref, sb_ref, out_ref):
    xc = jnp.clip(x_ref[:], -1.0, 1.0)
    out_ref[:] = (sp_ref[:]
                  + jnp.dot(xc, swt_ref[:], preferred_element_type=jnp.float32)
                  + sb_ref[:])


def kernel(x, values, skip_w, skip_b):
    B, D = x.shape
    O, _, K = values.shape
    NW = 32
    GCH = 64
    BT = 2560
    BS = B - BT
    NB = BS // NW

    swt = skip_w.T
    sb2 = skip_b[None, :]
    v3 = values.transpose(1, 2, 0)
    pt = _pack_table(v3)
    spline_sc = _sc_spline_fn(BS, D, O, K, NB, GCH, NB)(
        x[BT:].reshape(-1), pt)
    vt = values.reshape(O, D * K).T
    y_tc = _tc_spline(x[:BT], vt, swt, sb2, K)

    bm = min(BS, 512)
    y_sc = pl.pallas_call(
        _finish_body,
        grid=(BS // bm,),
        in_specs=[
            pl.BlockSpec((bm, O), lambda i: (i, 0)),
            pl.BlockSpec((bm, D), lambda i: (i, 0)),
            pl.BlockSpec((D, O), lambda i: (0, 0)),
            pl.BlockSpec((1, O), lambda i: (0, 0)),
        ],
        out_specs=pl.BlockSpec((bm, O), lambda i: (i, 0)),
        out_shape=jax.ShapeDtypeStruct((BS, O), jnp.float32),
    )(spline_sc, x[BT:], swt, sb2)
    return jnp.concatenate([y_tc, y_sc], axis=0)
